# Initial kernel scaffold; baseline (speedup 1.0000x reference)
import jax
import jax.numpy as jnp
from jax.experimental import pallas as pl


def kernel(features, edge_index, W1, b1, W2, b2, W3, b3, W4, b4, W5, b5, g1, bt1, g2, bt2):
    raise NotImplementedError("write your pallas kernel here")


# speedup vs baseline: 6.1328x; 6.1328x over previous
import functools

import jax
import jax.numpy as jnp
from jax import lax
from jax.experimental import pallas as pl
from jax.experimental.pallas import tpu as pltpu
from jax.experimental.pallas import tpu_sc as plsc

N = 10000
E = 320000
F = 128
NTILES = 32
EPT = E // NTILES
NB = 80
BK = 128
ACC_ROWS = N + 112
ROWS_PER_TILE = ACC_ROWS // 16
BN_BLK = 400
GRID = N // BN_BLK

_mesh = plsc.VectorSubcoreMesh(core_axis_name="c", subcore_axis_name="s")


def _make_deg_kernel():
    @functools.partial(
        pl.kernel,
        mesh=_mesh,
        compiler_params=pltpu.CompilerParams(needs_layout_passes=False),
        out_type=jax.ShapeDtypeStruct((NTILES, N), jnp.float32),
        scratch_types=[
            pltpu.VMEM((EPT,), jnp.int32),
            pltpu.VMEM((N,), jnp.float32),
        ],
    )
    def deg_kernel(dst_e, hout, dstv, hist):
        c = lax.axis_index("c")
        s = lax.axis_index("s")
        gid = c * 16 + s
        pltpu.sync_copy(dst_e.at[pl.ds(gid * EPT, EPT)], dstv)

        zro = jnp.zeros((16,), jnp.float32)

        def zbody(i, carry):
            hist[pl.ds(i * 16, 16)] = zro
            return carry

        lax.fori_loop(0, N // 16, zbody, 0)

        ones = jnp.ones((16,), jnp.float32)

        def hbody(i, carry):
            idx = dstv[pl.ds(i * 16, 16)]
            plsc.addupdate_scatter(hist, [idx], ones)
            return carry

        lax.fori_loop(0, EPT // 16, hbody, 0)
        pltpu.sync_copy(hist, hout.at[gid])

    return deg_kernel


def _make_agg_kernel(nc):
    @functools.partial(
        pl.kernel,
        mesh=_mesh,
        compiler_params=pltpu.CompilerParams(needs_layout_passes=False),
        out_type=jax.ShapeDtypeStruct((2, nc, N, F), jnp.float32),
        scratch_types=[
            pltpu.VMEM((NB, BK), jnp.int32),
            pltpu.VMEM((1, BK), jnp.int32),
            pltpu.VMEM((1, BK), jnp.int32),
            pltpu.VMEM((BK, F), jnp.float32),
            pltpu.VMEM((BK, F), jnp.float32),
            pltpu.VMEM_SHARED((ACC_ROWS, F), jnp.float32),
            pltpu.SemaphoreType.DMA,
            pltpu.SemaphoreType.DMA,
            pltpu.SemaphoreType.DMA,
            pltpu.SemaphoreType.DMA,
        ],
    )
    def agg_kernel(z, src_i, dst_i, zrs, part, srcv, db0, db1, gb0, gb1,
                   acc, sg0, sg1, sd0, sd1):
        c = lax.axis_index("c")
        s = lax.axis_index("s")
        gid = c * 16 + s
        pltpu.sync_copy(src_i.at[gid], srcv)
        drow = dst_i.at[gid]

        for cf in range(nc):
            zc = z.at[cf]
            off = s * ROWS_PER_TILE
            pltpu.sync_copy(zrs.at[pl.ds(off, ROWS_PER_TILE)],
                            acc.at[pl.ds(off, ROWS_PER_TILE)])
            plsc.subcore_barrier()

            pltpu.async_copy(drow.at[pl.ds(0, 1)], db0, sd0)
            pltpu.async_copy(zc.at[srcv.at[0]], gb0, sg0)

            def bat(j, carry):
                jj = 2 * j
                pltpu.async_copy(drow.at[pl.ds(jj + 1, 1)], db1, sd1)
                pltpu.async_copy(zc.at[srcv.at[jj + 1]], gb1, sg1)
                pltpu.make_async_copy(zc.at[srcv.at[jj]], gb0, sg0).wait()
                pltpu.make_async_copy(drow.at[pl.ds(jj, 1)], db0, sd0).wait()
                pltpu.sync_copy(gb0, acc.at[db0.at[0]], add=True)

                @pl.when(j < NB // 2 - 1)
                def _():
                    pltpu.async_copy(drow.at[pl.ds(jj + 2, 1)], db0, sd0)
                    pltpu.async_copy(zc.at[srcv.at[jj + 2]], gb0, sg0)

                pltpu.make_async_copy(zc.at[srcv.at[jj + 1]], gb1, sg1).wait()
                pltpu.make_async_copy(drow.at[pl.ds(jj + 1, 1)], db1,
                                      sd1).wait()
                pltpu.sync_copy(gb1, acc.at[db1.at[0]], add=True)
                return carry

            lax.fori_loop(0, NB // 2, bat, 0)
            plsc.subcore_barrier()

            @pl.when(s < 15)
            def _():
                pltpu.sync_copy(
                    acc.at[pl.ds(off, ROWS_PER_TILE)],
                    part.at[c, cf, pl.ds(off, ROWS_PER_TILE)])

            @pl.when(s == 15)
            def _():
                pltpu.sync_copy(
                    acc.at[pl.ds(15 * ROWS_PER_TILE, N - 15 * ROWS_PER_TILE)],
                    part.at[c, cf, pl.ds(15 * ROWS_PER_TILE,
                                         N - 15 * ROWS_PER_TILE)])

    return agg_kernel


def _dinv_from_hist(dh_ref):
    deg = jnp.sum(dh_ref[...], axis=1) + 1.0
    return lax.rsqrt(deg)


def _cat_chunks(x):
    return jnp.concatenate([x[i] for i in range(x.shape[0])], axis=1)


def _write_chunks(ref, zn, nco):
    for ci in range(nco):
        ref[ci] = zn[:, ci * F:(ci + 1) * F]


def _tc_pre(features, degh):
    def body(f_ref, dh_ref, o_ref):
        dinv = _dinv_from_hist(dh_ref)
        o_ref[0] = f_ref[...] * dinv[:, None]

    return pl.pallas_call(
        body,
        grid=(GRID,),
        in_specs=[
            pl.BlockSpec((BN_BLK, 128), lambda i: (i, 0)),
            pl.BlockSpec((BN_BLK, NTILES), lambda i: (i, 0)),
        ],
        out_specs=pl.BlockSpec((1, BN_BLK, 128), lambda i: (0, i, 0)),
        out_shape=jax.ShapeDtypeStruct((1, N, 128), jnp.float32),
    )(features, degh)


def _tc_matrelu(p, t, b, degh, w, nc, nco):
    dk1 = w.shape[1]
    dk = nc * F

    def body(p_ref, t_ref, b_ref, dh_ref, w_ref, o_ref):
        dinv = _dinv_from_hist(dh_ref)
        pb = p_ref[...]
        agg = _cat_chunks(pb[0] + pb[1] + t_ref[...])
        u = agg * dinv[:, None]
        h = jnp.maximum(
            jnp.dot(u, w_ref[...], preferred_element_type=jnp.float32, precision=lax.Precision.HIGHEST)
            + b_ref[...], 0.0)
        _write_chunks(o_ref, h * dinv[:, None], nco)

    return pl.pallas_call(
        body,
        grid=(GRID,),
        in_specs=[
            pl.BlockSpec((2, nc, BN_BLK, F), lambda i: (0, 0, i, 0)),
            pl.BlockSpec((nc, BN_BLK, F), lambda i: (0, i, 0)),
            pl.BlockSpec((1, dk1), lambda i: (0, 0)),
            pl.BlockSpec((BN_BLK, NTILES), lambda i: (i, 0)),
            pl.BlockSpec((dk, dk1), lambda i: (0, 0)),
        ],
        out_specs=pl.BlockSpec((nco, BN_BLK, F), lambda i: (0, i, 0)),
        out_shape=jax.ShapeDtypeStruct((nco, N, F), jnp.float32),
    )(p, t, b, degh, w)


def _tc_bn_a(p, t, b, degh, w, nc):
    dk1 = w.shape[1]
    dk = nc * F

    def body(p_ref, t_ref, b_ref, dh_ref, w_ref, y_ref, m_ref):
        dinv = _dinv_from_hist(dh_ref)
        pb = p_ref[...]
        agg = _cat_chunks(pb[0] + pb[1] + t_ref[...])
        u = agg * dinv[:, None]
        y = jnp.dot(u, w_ref[...],
                    preferred_element_type=jnp.float32, precision=lax.Precision.HIGHEST) + b_ref[...]
        y_ref[...] = y
        s1 = jnp.sum(y, axis=0, keepdims=True)
        s2 = jnp.sum(y * y, axis=0, keepdims=True)
        blk = jnp.concatenate([s1, s2, jnp.zeros((6, dk1), jnp.float32)],
                              axis=0)

        @pl.when(pl.program_id(0) == 0)
        def _():
            m_ref[...] = blk

        @pl.when(pl.program_id(0) > 0)
        def _():
            m_ref[...] += blk

    return pl.pallas_call(
        body,
        grid=(GRID,),
        in_specs=[
            pl.BlockSpec((2, nc, BN_BLK, F), lambda i: (0, 0, i, 0)),
            pl.BlockSpec((nc, BN_BLK, F), lambda i: (0, i, 0)),
            pl.BlockSpec((1, dk1), lambda i: (0, 0)),
            pl.BlockSpec((BN_BLK, NTILES), lambda i: (i, 0)),
            pl.BlockSpec((dk, dk1), lambda i: (0, 0)),
        ],
        out_specs=[
            pl.BlockSpec((BN_BLK, dk1), lambda i: (i, 0)),
            pl.BlockSpec((8, dk1), lambda i: (0, 0)),
        ],
        out_shape=[
            jax.ShapeDtypeStruct((N, dk1), jnp.float32),
            jax.ShapeDtypeStruct((8, dk1), jnp.float32),
        ],
    )(p, t, b, degh, w)


def _tc_bn_t(y, mom, g, bt, degh, nco):
    dk = y.shape[1]

    def body(y_ref, m_ref, g_ref, bt_ref, dh_ref, o_ref):
        dinv = _dinv_from_hist(dh_ref)
        mom_v = m_ref[...]
        mu = mom_v[0] * (1.0 / N)
        var = mom_v[1] * (1.0 / N) - mu * mu
        scale = lax.rsqrt(var + 1e-5) * g_ref[0]
        h = jnp.maximum((y_ref[...] - mu[None, :]) * scale[None, :]
                        + bt_ref[...], 0.0)
        _write_chunks(o_ref, h * dinv[:, None], nco)

    return pl.pallas_call(
        body,
        grid=(GRID,),
        in_specs=[
            pl.BlockSpec((BN_BLK, dk), lambda i: (i, 0)),
            pl.BlockSpec((8, dk), lambda i: (0, 0)),
            pl.BlockSpec((1, dk), lambda i: (0, 0)),
            pl.BlockSpec((1, dk), lambda i: (0, 0)),
            pl.BlockSpec((BN_BLK, NTILES), lambda i: (i, 0)),
        ],
        out_specs=pl.BlockSpec((nco, BN_BLK, F), lambda i: (0, i, 0)),
        out_shape=jax.ShapeDtypeStruct((nco, N, F), jnp.float32),
    )(y, mom, g, bt, degh)


def _tc_bn_b(y, mom, g, bt, degh, w, nco):
    dk = y.shape[1]
    dk1 = w.shape[1]

    def body(y_ref, m_ref, g_ref, bt_ref, dh_ref, w_ref, o_ref):
        dinv = _dinv_from_hist(dh_ref)
        mom_v = m_ref[...]
        mu = mom_v[0] * (1.0 / N)
        var = mom_v[1] * (1.0 / N) - mu * mu
        scale = lax.rsqrt(var + 1e-5) * g_ref[0]
        h = jnp.maximum((y_ref[...] - mu[None, :]) * scale[None, :]
                        + bt_ref[...], 0.0)
        zn = jnp.dot(h, w_ref[...],
                     preferred_element_type=jnp.float32, precision=lax.Precision.HIGHEST) * dinv[:, None]
        _write_chunks(o_ref, zn, nco)

    return pl.pallas_call(
        body,
        grid=(GRID,),
        in_specs=[
            pl.BlockSpec((BN_BLK, dk), lambda i: (i, 0)),
            pl.BlockSpec((8, dk), lambda i: (0, 0)),
            pl.BlockSpec((1, dk), lambda i: (0, 0)),
            pl.BlockSpec((1, dk), lambda i: (0, 0)),
            pl.BlockSpec((BN_BLK, NTILES), lambda i: (i, 0)),
            pl.BlockSpec((dk, dk1), lambda i: (0, 0)),
        ],
        out_specs=pl.BlockSpec((nco, BN_BLK, F), lambda i: (0, i, 0)),
        out_shape=jax.ShapeDtypeStruct((nco, N, F), jnp.float32),
    )(y, mom, g, bt, degh, w)


def _tc_last(p, z, b, degh, nc):
    dk = nc * F

    def body(p_ref, z_ref, b_ref, dh_ref, o_ref):
        dinv = _dinv_from_hist(dh_ref)
        pb = p_ref[...]
        agg = _cat_chunks(pb[0] + pb[1] + z_ref[...])
        o_ref[...] = jnp.maximum(agg * dinv[:, None] + b_ref[...], 0.0)

    return pl.pallas_call(
        body,
        grid=(GRID,),
        in_specs=[
            pl.BlockSpec((2, nc, BN_BLK, F), lambda i: (0, 0, i, 0)),
            pl.BlockSpec((nc, BN_BLK, F), lambda i: (0, i, 0)),
            pl.BlockSpec((1, dk), lambda i: (0, 0)),
            pl.BlockSpec((BN_BLK, NTILES), lambda i: (i, 0)),
        ],
        out_specs=pl.BlockSpec((BN_BLK, dk), lambda i: (i, 0)),
        out_shape=jax.ShapeDtypeStruct((N, dk), jnp.float32),
    )(p, z, b, degh)


def kernel(features, edge_index, W1, b1, W2, b2, W3, b3, W4, b4, W5, b5,
           g1, bt1, g2, bt2):
    pad = NB * BK - EPT
    src = edge_index[0].reshape(NTILES, EPT)
    dst = edge_index[1].reshape(NTILES, EPT)
    src_i = jnp.concatenate(
        [src, jnp.zeros((NTILES, pad), jnp.int32)], axis=1
    ).reshape(NTILES, NB, BK)
    dst_pad = (N + (jnp.arange(pad, dtype=jnp.int32) % 16))[None, :]
    dst_i = jnp.concatenate(
        [dst, jnp.broadcast_to(dst_pad, (NTILES, pad))], axis=1
    ).reshape(NTILES, NB, BK)
    zrs = jnp.zeros((ACC_ROWS, F), jnp.float32)

    deg_k = _make_deg_kernel()
    degh = deg_k(edge_index[1]).T

    agg1 = _make_agg_kernel(1)
    agg2 = _make_agg_kernel(2)
    agg4 = _make_agg_kernel(4)

    t0 = _tc_pre(features, degh)
    p = agg1(t0, src_i, dst_i, zrs)
    t1 = _tc_matrelu(p, t0, b1.reshape(1, -1), degh, W1, 1, 1)
    p = agg1(t1, src_i, dst_i, zrs)
    y2, mom = _tc_bn_a(p, t1, b2.reshape(1, -1), degh, W2, 1)
    t2 = _tc_bn_t(y2, mom, g1.reshape(1, -1), bt1.reshape(1, -1),
                  degh, 2)
    p = agg2(t2, src_i, dst_i, zrs)
    t3 = _tc_matrelu(p, t2, b3.reshape(1, -1), degh, W3, 2, 4)
    p = agg4(t3, src_i, dst_i, zrs)
    y4, mom2 = _tc_bn_a(p, t3, b4.reshape(1, -1), degh, W4, 4)
    z4 = _tc_bn_b(y4, mom2, g2.reshape(1, -1), bt2.reshape(1, -1), degh,
                  W5, 2)
    p = agg2(z4, src_i, dst_i, zrs)
    return _tc_last(p, z4, b5.reshape(1, -1), degh, 2)
